# baseline (speedup 1.0000x reference)
import functools

import jax
import jax.numpy as jnp
from jax.experimental import pallas as pl

_SMOOTHING = 0.1


def _bce_body(x_ref, tgt_ref, o_ref, *, nsteps, inv_n, on_value, off_value):
    i = pl.program_id(0)

    @pl.when(i == 0)
    def _init():
        o_ref[...] = jnp.zeros_like(o_ref)

    xb = x_ref[...]
    tgt = tgt_ref[...]
    col = jax.lax.broadcasted_iota(jnp.int32, xb.shape, 1)
    t = jnp.where(col == tgt, on_value, off_value)
    sp = jnp.maximum(xb, 0.0) + jnp.log1p(jnp.exp(-jnp.abs(xb)))
    o_ref[...] = o_ref[...] + jnp.sum(sp - xb * t)

    @pl.when(i == nsteps - 1)
    def _finish():
        o_ref[...] = o_ref[...] * inv_n


def kernel(x, target):
    b, c = x.shape
    off_value = _SMOOTHING / c
    on_value = 1.0 - _SMOOTHING + off_value
    tgt = target.reshape(b, 1).astype(jnp.int32)

    block_rows = 256
    nsteps = b // block_rows

    out = pl.pallas_call(
        functools.partial(
            _bce_body,
            nsteps=nsteps,
            inv_n=1.0 / (b * c),
            on_value=float(on_value),
            off_value=float(off_value),
        ),
        grid=(nsteps,),
        in_specs=[
            pl.BlockSpec((block_rows, c), lambda i: (i, 0)),
            pl.BlockSpec((block_rows, 1), lambda i: (i, 0)),
        ],
        out_specs=pl.BlockSpec((1, 1), lambda i: (0, 0)),
        out_shape=jax.ShapeDtypeStruct((1, 1), jnp.float32),
    )(x, tgt)
    return out[0, 0]


# speedup vs baseline: 1.0517x; 1.0517x over previous
import functools

import jax
import jax.numpy as jnp
from jax.experimental import pallas as pl

_SMOOTHING = 0.1


def _bce_body(x_ref, tgt_ref, o_ref, *, nsteps, inv_n, on_value, off_value):
    i = pl.program_id(0)

    @pl.when(i == 0)
    def _init():
        o_ref[...] = jnp.zeros_like(o_ref)

    xb = x_ref[...]
    tgt = tgt_ref[...]
    col = jax.lax.broadcasted_iota(jnp.int32, xb.shape, 1)
    t = jnp.where(col == tgt, on_value, off_value)
    sp = jnp.maximum(xb, 0.0) + jnp.log1p(jnp.exp(-jnp.abs(xb)))
    o_ref[...] = o_ref[...] + jnp.sum(sp - xb * t)

    @pl.when(i == nsteps - 1)
    def _finish():
        o_ref[...] = o_ref[...] * inv_n


def kernel(x, target):
    b, c = x.shape
    off_value = _SMOOTHING / c
    on_value = 1.0 - _SMOOTHING + off_value
    tgt = target.reshape(b, 1).astype(jnp.int32)

    block_rows = 1024
    nsteps = b // block_rows

    out = pl.pallas_call(
        functools.partial(
            _bce_body,
            nsteps=nsteps,
            inv_n=1.0 / (b * c),
            on_value=float(on_value),
            off_value=float(off_value),
        ),
        grid=(nsteps,),
        in_specs=[
            pl.BlockSpec((block_rows, c), lambda i: (i, 0)),
            pl.BlockSpec((block_rows, 1), lambda i: (i, 0)),
        ],
        out_specs=pl.BlockSpec((1, 1), lambda i: (0, 0)),
        out_shape=jax.ShapeDtypeStruct((1, 1), jnp.float32),
    )(x, tgt)
    return out[0, 0]


# speedup vs baseline: 1.2782x; 1.2154x over previous
import functools

import jax
import jax.numpy as jnp
from jax.experimental import pallas as pl
from jax.experimental.pallas import tpu as pltpu

_SMOOTHING = 0.1
_LOG2E = 1.4426950408889634
_LN2 = 0.6931471805599453


def _bce_body(x_ref, tgt_ref, o_ref, acc_ref, *, nsteps, inv_n, off_value,
              on_minus_off):
    i = pl.program_id(0)

    @pl.when(i == 0)
    def _init():
        acc_ref[0] = 0.0
        acc_ref[1] = 0.0
        acc_ref[2] = 0.0
        acc_ref[3] = 0.0

    xb = x_ref[...]
    tgt = tgt_ref[...]
    col = jax.lax.broadcasted_iota(jnp.int32, xb.shape, 1)
    a = jnp.abs(xb)
    l = jnp.log2(1.0 + jnp.exp2(a * (-_LOG2E)))
    g = jnp.where(col == tgt, xb, 0.0)
    acc_ref[0] += jnp.sum(l)
    acc_ref[1] += jnp.sum(a)
    acc_ref[2] += jnp.sum(xb)
    acc_ref[3] += jnp.sum(g)

    @pl.when(i == nsteps - 1)
    def _finish():
        total = (_LN2 * acc_ref[0] + 0.5 * acc_ref[1]
                 + (0.5 - off_value) * acc_ref[2] - on_minus_off * acc_ref[3])
        o_ref[0] = total * inv_n


def kernel(x, target):
    b, c = x.shape
    off_value = _SMOOTHING / c
    tgt = target.reshape(b, 1).astype(jnp.int32)

    block_rows = 1024
    nsteps = b // block_rows

    out = pl.pallas_call(
        functools.partial(
            _bce_body,
            nsteps=nsteps,
            inv_n=1.0 / (b * c),
            off_value=float(off_value),
            on_minus_off=float(1.0 - _SMOOTHING),
        ),
        grid=(nsteps,),
        in_specs=[
            pl.BlockSpec((block_rows, c), lambda i: (i, 0)),
            pl.BlockSpec((block_rows, 1), lambda i: (i, 0)),
        ],
        out_specs=pl.BlockSpec(memory_space=pltpu.SMEM),
        out_shape=jax.ShapeDtypeStruct((1,), jnp.float32),
        scratch_shapes=[pltpu.SMEM((4,), jnp.float32)],
    )(x, tgt)
    return out[0]
